# Initial kernel scaffold; baseline (speedup 1.0000x reference)
import functools

import jax
import jax.numpy as jnp
from jax import lax
from jax.experimental import pallas as pl
from jax.experimental.pallas import tpu as pltpu
from jax.experimental.pallas import tpu_sc as plsc

D = 128
K = 8
L = 16
NC, NS = 2, 16
NW = NC * NS
CHUNK = 28
N_CHUNKS = 56
B_PER_W = CHUNK * N_CHUNKS
M_PAD = B_PER_W * NW


def _pool_body(x_hbm, idx_hbm, out_hbm, idx_v, rows_v, out_v, g0, g1):
    wid = lax.axis_index("s") * NC + lax.axis_index("c")
    base = wid * B_PER_W

    pltpu.sync_copy(idx_hbm.at[wid], idx_v)

    gsems = (g0, g1)

    def start_gather(c, buf):
        pltpu.async_copy(x_hbm.at[idx_v.at[c]], rows_v.at[buf], gsems[buf])

    def wait_gather(c, buf):
        pltpu.make_async_copy(
            x_hbm.at[idx_v.at[c]], rows_v.at[buf], gsems[buf]
        ).wait()

    def compute_and_store(c, buf):
        rows = rows_v.at[buf]

        def row_body(r, carry):
            b = r * K
            for dcol in range(D // L):
                sl = pl.ds(dcol * L, L)
                acc = rows[b, sl]
                for k in range(1, K):
                    acc = jnp.maximum(acc, rows[b + k, sl])
                out_v[r, sl] = acc
            return carry

        lax.fori_loop(0, CHUNK, row_body, 0)
        pltpu.sync_copy(out_v, out_hbm.at[pl.ds(base + c * CHUNK, CHUNK)])

    start_gather(0, 0)

    def pair_body(g, carry):
        c0 = 2 * g
        start_gather(c0 + 1, 1)
        wait_gather(c0, 0)
        compute_and_store(c0, 0)

        @pl.when(g + 1 < N_CHUNKS // 2)
        def _():
            start_gather(c0 + 2, 0)

        wait_gather(c0 + 1, 1)
        compute_and_store(c0 + 1, 1)
        return carry

    lax.fori_loop(0, N_CHUNKS // 2, pair_body, 0)


def kernel(x, indices, weights):
    del weights
    m = indices.shape[0]
    idx = indices.astype(jnp.int32)
    idx = jnp.pad(idx, ((0, M_PAD - m), (0, 0)))
    idx = idx.reshape(NW, N_CHUNKS, CHUNK * K)

    mesh = plsc.VectorSubcoreMesh(core_axis_name="c", subcore_axis_name="s")
    out = pl.kernel(
        _pool_body,
        out_type=jax.ShapeDtypeStruct((M_PAD, D), jnp.float32),
        mesh=mesh,
        scratch_types=[
            pltpu.VMEM((N_CHUNKS, CHUNK * K), jnp.int32),
            pltpu.VMEM((2, CHUNK * K, D), jnp.float32),
            pltpu.VMEM((CHUNK, D), jnp.float32),
            pltpu.SemaphoreType.DMA,
            pltpu.SemaphoreType.DMA,
        ],
    )(x, idx)
    return out[:m]


# speedup vs baseline: 1.7603x; 1.7603x over previous
import functools

import jax
import jax.numpy as jnp
from jax import lax
from jax.experimental import pallas as pl
from jax.experimental.pallas import tpu as pltpu
from jax.experimental.pallas import tpu_sc as plsc

D = 128
K = 8
L = 16
NC, NS = 2, 16
NW = NC * NS
CHUNK = 16
N_CHUNKS = 100
B_PER_W = CHUNK * N_CHUNKS
M_PAD = B_PER_W * NW


def _pool_body(x_hbm, idx_hbm, out_hbm, idx_v, rows_v, out_v, g0, g1):
    wid = lax.axis_index("s") * NC + lax.axis_index("c")
    base = wid * B_PER_W

    pltpu.sync_copy(idx_hbm.at[wid], idx_v)

    gsems = (g0, g1)

    def start_gather(c, buf):
        pltpu.async_copy(x_hbm.at[idx_v.at[c]], rows_v.at[buf], gsems[buf])

    def wait_gather(c, buf):
        pltpu.make_async_copy(
            x_hbm.at[idx_v.at[c]], rows_v.at[buf], gsems[buf]
        ).wait()

    def compute_and_store(c, buf):
        rows = rows_v.at[buf]

        def row_body(r, carry):
            b = r * K
            for dcol in range(D // L):
                sl = pl.ds(dcol * L, L)
                acc = rows[b, sl]
                for k in range(1, K):
                    acc = jnp.maximum(acc, rows[b + k, sl])
                out_v[r, sl] = acc
            return carry

        lax.fori_loop(0, CHUNK, row_body, 0)
        pltpu.sync_copy(out_v, out_hbm.at[pl.ds(base + c * CHUNK, CHUNK)])

    start_gather(0, 0)

    def pair_body(g, carry):
        c0 = 2 * g
        start_gather(c0 + 1, 1)
        wait_gather(c0, 0)
        compute_and_store(c0, 0)

        @pl.when(g + 1 < N_CHUNKS // 2)
        def _():
            start_gather(c0 + 2, 0)

        wait_gather(c0 + 1, 1)
        compute_and_store(c0 + 1, 1)
        return carry

    lax.fori_loop(0, N_CHUNKS // 2, pair_body, 0)


def kernel(x, indices, weights):
    del weights
    m = indices.shape[0]
    idx = indices.astype(jnp.int32)
    idx = jnp.pad(idx, ((0, M_PAD - m), (0, 0)))
    idx = idx.reshape(NW, N_CHUNKS, CHUNK * K)

    mesh = plsc.VectorSubcoreMesh(core_axis_name="c", subcore_axis_name="s")
    out = pl.kernel(
        _pool_body,
        out_type=jax.ShapeDtypeStruct((M_PAD, D), jnp.float32),
        mesh=mesh,
        scratch_types=[
            pltpu.VMEM((N_CHUNKS, CHUNK * K), jnp.int32),
            pltpu.VMEM((2, CHUNK * K, D), jnp.float32),
            pltpu.VMEM((CHUNK, D), jnp.float32),
            pltpu.SemaphoreType.DMA,
            pltpu.SemaphoreType.DMA,
        ],
    )(x, idx)
    return out[:m]


# speedup vs baseline: 1.7933x; 1.0188x over previous
import functools

import jax
import jax.numpy as jnp
from jax import lax
from jax.experimental import pallas as pl
from jax.experimental.pallas import tpu as pltpu
from jax.experimental.pallas import tpu_sc as plsc

D = 128
K = 8
L = 16
NC, NS = 2, 16
NW = NC * NS
CHUNK = 16
N_CHUNKS = 100
B_PER_W = CHUNK * N_CHUNKS
M_PAD = B_PER_W * NW


NBUF = 4


def _pool_body(x_hbm, idx_hbm, out_hbm, idx_v, rows_v, out_v, *gsems):
    wid = lax.axis_index("s") * NC + lax.axis_index("c")
    base = wid * B_PER_W

    pltpu.sync_copy(idx_hbm.at[wid], idx_v)

    def start_gather(c, buf):
        pltpu.async_copy(x_hbm.at[idx_v.at[c]], rows_v.at[buf], gsems[buf])

    def wait_gather(c, buf):
        pltpu.make_async_copy(
            x_hbm.at[idx_v.at[c]], rows_v.at[buf], gsems[buf]
        ).wait()

    def compute_and_store(c, buf):
        rows = rows_v.at[buf]

        def row_body(r, carry):
            b = r * K
            for dcol in range(D // L):
                sl = pl.ds(dcol * L, L)
                acc = rows[b, sl]
                for k in range(1, K):
                    acc = jnp.maximum(acc, rows[b + k, sl])
                out_v[r, sl] = acc
            return carry

        lax.fori_loop(0, CHUNK, row_body, 0)
        pltpu.sync_copy(out_v, out_hbm.at[pl.ds(base + c * CHUNK, CHUNK)])

    for b in range(NBUF - 1):
        start_gather(b, b)

    def group_body(g, carry):
        c0 = NBUF * g
        for b in range(NBUF):
            c = c0 + b
            wait_gather(c, b)

            @pl.when(c + NBUF - 1 < N_CHUNKS)
            def _():
                start_gather(c + NBUF - 1, (b + NBUF - 1) % NBUF)

            compute_and_store(c, b)
        return carry

    lax.fori_loop(0, N_CHUNKS // NBUF, group_body, 0)


def kernel(x, indices, weights):
    del weights
    m = indices.shape[0]
    idx = indices.astype(jnp.int32)
    idx = jnp.pad(idx, ((0, M_PAD - m), (0, 0)))
    idx = idx.reshape(NW, N_CHUNKS, CHUNK * K)

    mesh = plsc.VectorSubcoreMesh(core_axis_name="c", subcore_axis_name="s")
    out = pl.kernel(
        _pool_body,
        out_type=jax.ShapeDtypeStruct((M_PAD, D), jnp.float32),
        mesh=mesh,
        scratch_types=[
            pltpu.VMEM((N_CHUNKS, CHUNK * K), jnp.int32),
            pltpu.VMEM((NBUF, CHUNK * K, D), jnp.float32),
            pltpu.VMEM((CHUNK, D), jnp.float32),
        ] + [pltpu.SemaphoreType.DMA] * NBUF,
    )(x, idx)
    return out[:m]


# speedup vs baseline: 1.8006x; 1.0040x over previous
import functools

import jax
import jax.numpy as jnp
from jax import lax
from jax.experimental import pallas as pl
from jax.experimental.pallas import tpu as pltpu
from jax.experimental.pallas import tpu_sc as plsc

D = 128
K = 8
L = 16
NC, NS = 2, 16
NW = NC * NS
CHUNK = 16
CH_FAST = 160
CH_SLOW = 40
N_CHUNKS_TOTAL = NS * (CH_FAST + CH_SLOW)
M_PAD = N_CHUNKS_TOTAL * CHUNK
FAST_CORE = 0


NBUF = 4


def _pool_body(x_hbm, idx_hbm, out_hbm, idx_v, rows_v, out_v, *gsems):
    c_ax = lax.axis_index("c")
    s_ax = lax.axis_index("s")

    def run(base_chunk, n_chunks):
        pltpu.sync_copy(
            idx_hbm.at[pl.ds(base_chunk, n_chunks)],
            idx_v.at[pl.ds(0, n_chunks)],
        )
        base_row = base_chunk * CHUNK

        def start_gather(c, buf):
            pltpu.async_copy(x_hbm.at[idx_v.at[c]], rows_v.at[buf], gsems[buf])

        def wait_gather(c, buf):
            pltpu.make_async_copy(
                x_hbm.at[idx_v.at[c]], rows_v.at[buf], gsems[buf]
            ).wait()

        def compute_and_store(c, buf):
            rows = rows_v.at[buf]

            def row_body(r, carry):
                b = r * K
                for dcol in range(D // L):
                    sl = pl.ds(dcol * L, L)
                    acc = rows[b, sl]
                    for k in range(1, K):
                        acc = jnp.maximum(acc, rows[b + k, sl])
                    out_v[r, sl] = acc
                return carry

            lax.fori_loop(0, CHUNK, row_body, 0)
            pltpu.sync_copy(
                out_v, out_hbm.at[pl.ds(base_row + c * CHUNK, CHUNK)]
            )

        for b in range(NBUF - 1):
            start_gather(b, b)

        def group_body(g, carry):
            c0 = NBUF * g
            for b in range(NBUF):
                c = c0 + b
                wait_gather(c, b)

                @pl.when(c + NBUF - 1 < n_chunks)
                def _():
                    start_gather(c + NBUF - 1, (b + NBUF - 1) % NBUF)

                compute_and_store(c, b)
            return carry

        lax.fori_loop(0, n_chunks // NBUF, group_body, 0)

    @pl.when(c_ax == FAST_CORE)
    def _():
        run(s_ax * CH_FAST, CH_FAST)

    @pl.when(c_ax != FAST_CORE)
    def _():
        run(NS * CH_FAST + s_ax * CH_SLOW, CH_SLOW)


def kernel(x, indices, weights):
    del weights
    m = indices.shape[0]
    idx = indices.astype(jnp.int32)
    idx = jnp.pad(idx, ((0, M_PAD - m), (0, 0)))
    idx = idx.reshape(N_CHUNKS_TOTAL, CHUNK * K)

    mesh = plsc.VectorSubcoreMesh(core_axis_name="c", subcore_axis_name="s")
    out = pl.kernel(
        _pool_body,
        out_type=jax.ShapeDtypeStruct((M_PAD, D), jnp.float32),
        mesh=mesh,
        scratch_types=[
            pltpu.VMEM((CH_FAST, CHUNK * K), jnp.int32),
            pltpu.VMEM((NBUF, CHUNK * K, D), jnp.float32),
            pltpu.VMEM((CHUNK, D), jnp.float32),
        ] + [pltpu.SemaphoreType.DMA] * NBUF,
    )(x, idx)
    return out[:m]


# speedup vs baseline: 5.4284x; 3.0148x over previous
import functools

import jax
import jax.numpy as jnp
from jax import lax
from jax.experimental import pallas as pl
from jax.experimental.pallas import tpu as pltpu
from jax.experimental.pallas import tpu_sc as plsc

D = 128
K = 8
L = 16
NC, NS = 2, 16
NW = NC * NS
CHUNK = 16
CH_FAST = 104
CH_SLOW = 96
N_CHUNKS_TOTAL = 3200
M_PAD = N_CHUNKS_TOTAL * CHUNK
FAST_CORE = 0


NBUF = 4


def _pool_body(x_hbm, idx_hbm, out_hbm, idx_v, rows_v, out_v, *gsems):
    c_ax = lax.axis_index("c")
    s_ax = lax.axis_index("s")

    def run(base_chunk, n_chunks):
        pltpu.sync_copy(
            idx_hbm.at[pl.ds(base_chunk, n_chunks)],
            idx_v.at[pl.ds(0, n_chunks)],
        )
        base_row = base_chunk * CHUNK

        def start_gather(c, buf):
            pltpu.async_copy(x_hbm.at[idx_v.at[c]], rows_v.at[buf], gsems[buf])

        def wait_gather(c, buf):
            pltpu.make_async_copy(
                x_hbm.at[idx_v.at[c]], rows_v.at[buf], gsems[buf]
            ).wait()

        def compute_and_store(c, buf):
            rows = rows_v.at[buf]

            def row_body(r, carry):
                b = r * K
                for dcol in range(D // L):
                    sl = pl.ds(dcol * L, L)
                    acc = rows[b, sl]
                    for k in range(1, K):
                        acc = jnp.maximum(acc, rows[b + k, sl])
                    out_v[r, sl] = acc
                return carry

            lax.fori_loop(0, CHUNK, row_body, 0)
            pltpu.sync_copy(
                out_v, out_hbm.at[pl.ds(base_row + c * CHUNK, CHUNK)]
            )

        for b in range(NBUF - 1):
            start_gather(b, b)

        def group_body(g, carry):
            c0 = NBUF * g
            for b in range(NBUF):
                c = c0 + b
                wait_gather(c, b)

                @pl.when(c + NBUF - 1 < n_chunks)
                def _():
                    start_gather(c + NBUF - 1, (b + NBUF - 1) % NBUF)

                compute_and_store(c, b)
            return carry

        lax.fori_loop(0, n_chunks // NBUF, group_body, 0)

    @pl.when(c_ax == FAST_CORE)
    def _():
        run(s_ax * CH_FAST, CH_FAST)

    if CH_SLOW:
        @pl.when(c_ax != FAST_CORE)
        def _():
            run(NS * CH_FAST + s_ax * CH_SLOW, CH_SLOW)


def kernel(x, indices, weights):
    del weights
    m = indices.shape[0]
    idx = indices.astype(jnp.int32)
    n_pad = M_PAD - m
    v = x.shape[0]
    pad_idx = (jnp.arange(n_pad * K, dtype=jnp.int32) % v).reshape(n_pad, K)
    idx = jnp.concatenate([idx, pad_idx], axis=0)
    idx = idx.reshape(N_CHUNKS_TOTAL, CHUNK * K)

    mesh = plsc.VectorSubcoreMesh(core_axis_name="c", subcore_axis_name="s")
    out = pl.kernel(
        _pool_body,
        out_type=jax.ShapeDtypeStruct((M_PAD, D), jnp.float32),
        mesh=mesh,
        scratch_types=[
            pltpu.VMEM((CH_FAST, CHUNK * K), jnp.int32),
            pltpu.VMEM((NBUF, CHUNK * K, D), jnp.float32),
            pltpu.VMEM((CHUNK, D), jnp.float32),
        ] + [pltpu.SemaphoreType.DMA] * NBUF,
    )(x, idx)
    return out[:m]


# speedup vs baseline: 6.3138x; 1.1631x over previous
import jax
import jax.numpy as jnp
from jax import lax
from jax.experimental import pallas as pl
from jax.experimental.pallas import tpu as pltpu
from jax.experimental.pallas import tpu_sc as plsc

D = 128
K = 8
L = 16
NC, NS = 2, 16
NW = NC * NS
CHUNK = 16
NBUF = 4
CH_W = 98
N_CHUNKS_TOTAL = NW * CH_W
M_PAD = N_CHUNKS_TOTAL * CHUNK
IDX_PER_CHUNK = CHUNK * K


def _pool_body(x_hbm, idx_hbm, out_hbm, idx_v, rows_v, out_v, *gsems):
    wid = lax.axis_index("s") * NC + lax.axis_index("c")
    base_chunk = wid * CH_W
    base_row = base_chunk * CHUNK
    m_out = out_hbm.shape[0]
    n_real_chunks = m_out // CHUNK

    pltpu.sync_copy(
        idx_hbm.at[pl.ds(base_chunk * IDX_PER_CHUNK, CH_W * IDX_PER_CHUNK)],
        idx_v,
    )

    def start_gather(c, buf):
        pltpu.async_copy(
            x_hbm.at[idx_v.at[pl.ds(c * IDX_PER_CHUNK, IDX_PER_CHUNK)]],
            rows_v.at[buf],
            gsems[buf],
        )

    def wait_gather(c, buf):
        pltpu.make_async_copy(
            x_hbm.at[idx_v.at[pl.ds(c * IDX_PER_CHUNK, IDX_PER_CHUNK)]],
            rows_v.at[buf],
            gsems[buf],
        ).wait()

    def compute_and_store(c, buf):
        rows = rows_v.at[buf]

        def row_body(r, carry):
            b = r * K
            for dcol in range(D // L):
                sl = pl.ds(dcol * L, L)
                acc = rows[b, sl]
                for k in range(1, K):
                    acc = jnp.maximum(acc, rows[b + k, sl])
                out_v[r, sl] = acc
            return carry

        lax.fori_loop(0, CHUNK, row_body, 0)

        @pl.when(base_chunk + c < n_real_chunks)
        def _():
            pltpu.sync_copy(
                out_v, out_hbm.at[pl.ds(base_row + c * CHUNK, CHUNK)]
            )

    for b in range(NBUF - 1):
        start_gather(b, b)

    n_groups = CH_W // NBUF

    def group_body(g, carry):
        c0 = NBUF * g
        for b in range(NBUF):
            c = c0 + b
            wait_gather(c, b)

            @pl.when(c + NBUF - 1 < CH_W)
            def _():
                start_gather(c + NBUF - 1, (b + NBUF - 1) % NBUF)

            compute_and_store(c, b)
        return carry

    lax.fori_loop(0, n_groups, group_body, 0)

    for t in range(n_groups * NBUF, CH_W):
        wait_gather(t, t % NBUF)
        compute_and_store(t, t % NBUF)


def kernel(x, indices, weights):
    del weights
    m = indices.shape[0]
    v = x.shape[0]
    flat = jnp.ravel(indices.astype(jnp.int32))
    n_pad = M_PAD * K - flat.shape[0]
    pad_idx = jnp.arange(n_pad, dtype=jnp.int32) % v
    idx_flat = jnp.concatenate([flat, pad_idx])

    mesh = plsc.VectorSubcoreMesh(core_axis_name="c", subcore_axis_name="s")
    out = pl.kernel(
        _pool_body,
        out_type=jax.ShapeDtypeStruct((m, D), jnp.float32),
        mesh=mesh,
        scratch_types=[
            pltpu.VMEM((CH_W * IDX_PER_CHUNK,), jnp.int32),
            pltpu.VMEM((NBUF, IDX_PER_CHUNK, D), jnp.float32),
            pltpu.VMEM((CHUNK, D), jnp.float32),
        ] + [pltpu.SemaphoreType.DMA] * NBUF,
    )(x, idx_flat)
    return out
